# Initial kernel scaffold; baseline (speedup 1.0000x reference)
import jax
import jax.numpy as jnp
from jax.experimental import pallas as pl


def kernel(x):
    raise NotImplementedError("write your pallas kernel here")


# speedup vs baseline: 3.4845x; 3.4845x over previous
import functools

import jax
import jax.numpy as jnp
from jax import lax
from jax.experimental import pallas as pl
from jax.experimental.pallas import tpu as pltpu
from jax.experimental.pallas import tpu_sc as plsc

B = 64
N = 8192
K = 4096
L = 16
NC = 2
NS = 16
NW = NC * NS
ROWS_PER_W = B // NW
NV = N // L
TOPBIT = jnp.int32(-(2**31))


def _sc_body(x_hbm, out_hbm, xrow, outrow, keys, cand, hist):
    wid = lax.axis_index("s") * NC + lax.axis_index("c")
    iota = lax.broadcasted_iota(jnp.int32, (L,), 0)
    lane_base = iota * 256
    zeros_i = jnp.zeros((L,), jnp.int32)
    ones_i = jnp.ones((L,), jnp.int32)
    sh24 = jnp.full((L,), 24, jnp.int32)

    for r in range(ROWS_PER_W):
        row = wid * ROWS_PER_W + r
        pltpu.sync_copy(x_hbm.at[row], xrow)

        def zbody(j, c):
            plsc.store_scatter(hist, [j * L + iota], zeros_i)
            return c
        lax.fori_loop(0, (L * 256) // L, zbody, jnp.int32(0))

        def hbody(i, c):
            idx = i * L + iota
            xv = plsc.load_gather(xrow, [idx])
            bits = plsc.bitcast(xv, jnp.int32)
            key = jnp.where(bits < 0, ~bits, bits ^ TOPBIT)
            plsc.store_scatter(keys, [idx], key)
            bucket = lax.shift_right_logical(key, sh24)
            plsc.addupdate_scatter(hist, [lane_base + bucket], ones_i)
            return c
        lax.fori_loop(0, NV, hbody, jnp.int32(0))

        def sbody(i, carry):
            cum, b_star, above = carry
            v = 15 - i
            acc = zeros_i
            for l in range(L):
                acc = acc + plsc.load_gather(hist, [l * 256 + v * L + iota])
            s = lax.rev(jnp.cumsum(lax.rev(acc, (0,))), (0,))
            cnt_ge = cum + s
            mask = cnt_ge >= K
            b_loc = jnp.max(jnp.where(mask, v * L + iota, -1))
            a_loc = jnp.min(jnp.where(mask, cnt_ge - acc, jnp.int32(2**31 - 1)))
            better = b_loc > b_star
            b_star = jnp.where(better, b_loc, b_star)
            above = jnp.where(better, a_loc, above)
            cum = cum + jnp.sum(acc)
            return cum, b_star, above
        _, b1, above = lax.fori_loop(
            0, 16, sbody, (jnp.int32(0), jnp.int32(-1), jnp.int32(0)))
        kk = jnp.int32(K) - above

        def cbody(i, off):
            idx = i * L + iota
            kv = plsc.load_gather(keys, [idx])
            bucket = lax.shift_right_logical(kv, sh24)
            mk = bucket == b1
            pos = jnp.maximum(off + jnp.cumsum(mk.astype(jnp.int32)) - 1, 0)
            plsc.store_scatter(cand, [pos], kv, mask=mk)
            return off + plsc.all_reduce_population_count(mk)
        off = lax.fori_loop(0, NV, cbody, zeros_i)
        m = jnp.max(off)
        nvc = (m + L - 1) // L

        def bitbody(i, t):
            bit = 23 - i
            tp = lax.bitwise_or(t, lax.shift_left(jnp.int32(1), bit))
            def cnt_body(j, c):
                idx = j * L + iota
                kv = plsc.load_gather(cand, [idx])
                ge = jnp.logical_and(kv >= tp, idx < m)
                return c + plsc.all_reduce_population_count(ge)
            cnt = lax.fori_loop(0, nvc, cnt_body, zeros_i)
            return jnp.where(jnp.max(cnt) >= kk, tp, t)
        tkey = lax.fori_loop(0, 24, bitbody, lax.shift_left(b1, jnp.int32(24)))

        tsplat = zeros_i + tkey
        fbits = jnp.where(tsplat < 0, tsplat ^ TOPBIT, ~tsplat)
        tvec = plsc.bitcast(fbits, jnp.float32)

        def mbody(i, acc):
            idx = i * L + iota
            xv = plsc.load_gather(xrow, [idx])
            rv = jnp.where(xv >= tvec, xv, jnp.float32(0))
            plsc.store_scatter(outrow, [idx], rv)
            return acc + rv
        acc = lax.fori_loop(0, NV, mbody, jnp.zeros((L,), jnp.float32))
        s_splat = jnp.zeros((L,), jnp.float32) + jnp.sum(acc)
        scale = jnp.full((L,), N, jnp.float32) / s_splat

        def obody(i, c):
            idx = i * L + iota
            rv = plsc.load_gather(outrow, [idx])
            plsc.store_scatter(outrow, [idx], rv * scale)
            return c
        lax.fori_loop(0, NV, obody, jnp.int32(0))
        pltpu.sync_copy(outrow, out_hbm.at[row])


def kernel(x):
    mesh = plsc.VectorSubcoreMesh(core_axis_name="c", subcore_axis_name="s")
    f = functools.partial(
        pl.kernel,
        mesh=mesh,
        compiler_params=pltpu.CompilerParams(needs_layout_passes=False),
        out_type=jax.ShapeDtypeStruct((B, N), jnp.float32),
        scratch_types=[
            pltpu.VMEM((N,), jnp.float32),
            pltpu.VMEM((N,), jnp.float32),
            pltpu.VMEM((N,), jnp.int32),
            pltpu.VMEM((N,), jnp.int32),
            pltpu.VMEM((L * 256,), jnp.int32),
        ],
    )(_sc_body)
    return f(x)


# speedup vs baseline: 6.8476x; 1.9652x over previous
import functools

import jax
import jax.numpy as jnp
from jax import lax
from jax.experimental import pallas as pl
from jax.experimental.pallas import tpu as pltpu
from jax.experimental.pallas import tpu_sc as plsc

B = 64
N = 8192
K = 4096
L = 16
NC = 2
NS = 16
NW = NC * NS
ROWS_PER_W = B // NW
TOPBIT = jnp.int32(-(2**31))


def _sc_body(x_hbm, out_hbm, xrow, outrow, keys, cand, hist):
    wid = lax.axis_index("s") * NC + lax.axis_index("c")
    iota = lax.broadcasted_iota(jnp.int32, (L,), 0)
    lane_base = iota * 256
    zeros_i = jnp.zeros((L,), jnp.int32)
    ones_i = jnp.ones((L,), jnp.int32)
    sh24 = jnp.full((L,), 24, jnp.int32)

    for r in range(ROWS_PER_W):
        row = wid * ROWS_PER_W + r
        pltpu.sync_copy(x_hbm.at[row], xrow)

        @plsc.parallel_loop(0, L * 256, step=L, unroll=8)
        def _(j):
            hist[pl.ds(j, L)] = zeros_i

        @plsc.parallel_loop(0, N, step=L, unroll=8)
        def _(i):
            xv = xrow[pl.ds(i, L)]
            bits = plsc.bitcast(xv, jnp.int32)
            key = jnp.where(bits < 0, ~bits, bits ^ TOPBIT)
            keys[pl.ds(i, L)] = key
            bucket = lax.shift_right_logical(key, sh24)
            plsc.addupdate_scatter(hist, [lane_base + bucket], ones_i)

        def sbody(i, carry):
            cum, b_star, above = carry
            v = 15 - i
            acc = zeros_i
            for l in range(L):
                acc = acc + hist[pl.ds(l * 256 + v * L, L)]
            s = lax.rev(jnp.cumsum(lax.rev(acc, (0,))), (0,))
            cnt_ge = cum + s
            mask = cnt_ge >= K
            b_loc = jnp.max(jnp.where(mask, v * L + iota, -1))
            a_loc = jnp.min(jnp.where(mask, cnt_ge - acc, jnp.int32(2**31 - 1)))
            better = b_loc > b_star
            b_star = jnp.where(better, b_loc, b_star)
            above = jnp.where(better, a_loc, above)
            cum = cum + jnp.sum(acc)
            return cum, b_star, above
        _, b1, above = lax.fori_loop(
            0, 16, sbody, (jnp.int32(0), jnp.int32(-1), jnp.int32(0)))
        kk = jnp.int32(K) - above

        @plsc.parallel_loop(0, N, step=L, unroll=4, carry=zeros_i)
        def off(i, off):
            kv = keys[pl.ds(i, L)]
            bucket = lax.shift_right_logical(kv, sh24)
            mk = bucket == b1
            pos = jnp.maximum(off + jnp.cumsum(mk.astype(jnp.int32)) - 1, 0)
            plsc.store_scatter(cand, [pos], kv, mask=mk)
            return off + plsc.all_reduce_population_count(mk)
        m = jnp.max(off)

        def bitbody(i, t):
            bit = 23 - i
            tp = lax.bitwise_or(t, lax.shift_left(jnp.int32(1), bit))
            def cnt_body(j, c):
                kv = cand[pl.ds(j * L, L)]
                ge = jnp.logical_and(kv >= tp, j * L + iota < m)
                return c + plsc.all_reduce_population_count(ge)
            cnt = lax.fori_loop(0, (m + L - 1) // L, cnt_body, zeros_i)
            return jnp.where(jnp.max(cnt) >= kk, tp, t)
        tkey = lax.fori_loop(0, 24, bitbody, lax.shift_left(b1, jnp.int32(24)))

        tsplat = zeros_i + tkey
        fbits = jnp.where(tsplat < 0, tsplat ^ TOPBIT, ~tsplat)
        tvec = plsc.bitcast(fbits, jnp.float32)

        @plsc.parallel_loop(0, N, step=L, unroll=8,
                            carry=jnp.zeros((L,), jnp.float32))
        def acc(i, a):
            xv = xrow[pl.ds(i, L)]
            rv = jnp.where(xv >= tvec, xv, jnp.float32(0))
            outrow[pl.ds(i, L)] = rv
            return a + rv

        s_splat = jnp.zeros((L,), jnp.float32) + jnp.sum(acc)
        scale = jnp.full((L,), N, jnp.float32) / s_splat

        @plsc.parallel_loop(0, N, step=L, unroll=8)
        def _(i):
            outrow[pl.ds(i, L)] = outrow[pl.ds(i, L)] * scale

        pltpu.sync_copy(outrow, out_hbm.at[row])


def kernel(x):
    mesh = plsc.VectorSubcoreMesh(core_axis_name="c", subcore_axis_name="s")
    f = functools.partial(
        pl.kernel,
        mesh=mesh,
        compiler_params=pltpu.CompilerParams(needs_layout_passes=False),
        out_type=jax.ShapeDtypeStruct((B, N), jnp.float32),
        scratch_types=[
            pltpu.VMEM((N,), jnp.float32),
            pltpu.VMEM((N,), jnp.float32),
            pltpu.VMEM((N,), jnp.int32),
            pltpu.VMEM((N,), jnp.int32),
            pltpu.VMEM((L * 256,), jnp.int32),
        ],
    )(_sc_body)
    return f(x)
